# Initial kernel scaffold; baseline (speedup 1.0000x reference)
import jax
import jax.numpy as jnp
from jax.experimental import pallas as pl


def kernel(x, W_gate, b_gate, W_up, b_up, W_down, b_down):
    raise NotImplementedError("write your pallas kernel here")


# speedup vs baseline: 2.0927x; 2.0927x over previous
import jax
import jax.numpy as jnp
from jax.experimental import pallas as pl


def _ffn_body(x_ref, wg_ref, bg_ref, wu_ref, bu_ref, wd_ref, bd_ref,
              out_ref, gate_ref, *, ts):
    xb = x_ref[...]
    logits = jnp.dot(xb, wg_ref[...],
                     preferred_element_type=jnp.float32) + bg_ref[...]
    lane = jax.lax.broadcasted_iota(jnp.int32, logits.shape, 1)
    mx = jnp.max(logits, axis=-1, keepdims=True)
    num_e = logits.shape[-1]
    idx = jnp.min(jnp.where(logits == mx, lane, num_e), axis=-1,
                  keepdims=True)
    onehot = (lane == idx).astype(jnp.float32)
    gate_ref[...] = onehot
    h = jnp.dot(xb, wu_ref[...],
                preferred_element_type=jnp.float32) + bu_ref[...]
    tile_of_feat = jax.lax.broadcasted_iota(jnp.int32, h.shape, 1) // ts
    h = jnp.where(tile_of_feat == idx, h, 0.0)
    h = jnp.maximum(h, 0.0)
    out_ref[...] = jnp.dot(h, wd_ref[...],
                           preferred_element_type=jnp.float32) + bd_ref[...]


def kernel(x, W_gate, b_gate, W_up, b_up, W_down, b_down):
    B, T, C = x.shape
    N = B * T
    E = W_gate.shape[1]
    F = W_up.shape[1]
    TS = F // E
    x_f = x.reshape(N, C)
    BM = min(256, N)

    import functools
    body = functools.partial(_ffn_body, ts=TS)

    out, gate = pl.pallas_call(
        body,
        grid=(N // BM,),
        in_specs=[
            pl.BlockSpec((BM, C), lambda i: (i, 0)),
            pl.BlockSpec((C, E), lambda i: (0, 0)),
            pl.BlockSpec((1, E), lambda i: (0, 0)),
            pl.BlockSpec((C, F), lambda i: (0, 0)),
            pl.BlockSpec((1, F), lambda i: (0, 0)),
            pl.BlockSpec((F, C), lambda i: (0, 0)),
            pl.BlockSpec((1, C), lambda i: (0, 0)),
        ],
        out_specs=[
            pl.BlockSpec((BM, C), lambda i: (i, 0)),
            pl.BlockSpec((BM, E), lambda i: (i, 0)),
        ],
        out_shape=[
            jax.ShapeDtypeStruct((N, C), jnp.float32),
            jax.ShapeDtypeStruct((N, E), jnp.float32),
        ],
    )(x_f, W_gate, b_gate.reshape(1, E), W_up, b_up.reshape(1, F),
      W_down, b_down.reshape(1, C))
    return out.reshape(B, T, C), gate.reshape(B, T, E)
